# baseline (speedup 1.0000x reference)
import jax
from jax.experimental import pallas as pl
from jax.experimental.pallas import tpu as pltpu


def _dma_copy_kernel(x_hbm, o_hbm, sem):
    pltpu.make_async_copy(x_hbm, o_hbm, sem).start()
    pltpu.make_async_copy(x_hbm, o_hbm, sem).wait()


def kernel(x):
    B, T, F = x.shape
    rows = B * T
    x2 = x.reshape(rows, F)
    out = pl.pallas_call(
        _dma_copy_kernel,
        in_specs=[pl.BlockSpec(memory_space=pl.ANY)],
        out_specs=pl.BlockSpec(memory_space=pl.ANY),
        out_shape=jax.ShapeDtypeStruct((rows, F), x.dtype),
        scratch_shapes=[pltpu.SemaphoreType.DMA],
    )(x2)
    return out.reshape(B, T, F)


# speedup vs baseline: 29.5130x; 29.5130x over previous
import functools

import jax
import jax.numpy as jnp
from jax import lax
from jax.experimental import pallas as pl
from jax.experimental.pallas import tpu as pltpu
from jax.experimental.pallas import tpu_sc as plsc


def kernel(x):
    B, T, F = x.shape
    rows = B * T
    x2 = x.reshape(rows, F)

    info = plsc.get_sparse_core_info()
    NC, NS = info.num_cores, info.num_subcores
    NW = NC * NS
    rpw = rows // NW
    chunk = 256
    n_chunks = rpw // chunk

    mesh = plsc.VectorSubcoreMesh(core_axis_name="c", subcore_axis_name="s")

    @functools.partial(
        pl.kernel,
        mesh=mesh,
        out_type=jax.ShapeDtypeStruct((rows, F), x.dtype),
        scratch_types=[pltpu.VMEM((chunk, F), jnp.float32)],
    )
    def sc_copy(x_hbm, out_hbm, buf):
        wid = lax.axis_index("s") * NC + lax.axis_index("c")
        base = wid * rpw

        def body(i, carry):
            off = base + i * chunk
            pltpu.sync_copy(x_hbm.at[pl.ds(off, chunk), :], buf)
            pltpu.sync_copy(buf, out_hbm.at[pl.ds(off, chunk), :])
            return carry

        lax.fori_loop(0, n_chunks, body, 0)

    return sc_copy(x2).reshape(B, T, F)


# speedup vs baseline: 30.0530x; 1.0183x over previous
import functools

import jax
import jax.numpy as jnp
from jax import lax
from jax.experimental import pallas as pl
from jax.experimental.pallas import tpu as pltpu
from jax.experimental.pallas import tpu_sc as plsc


def kernel(x):
    B, T, F = x.shape
    rows = B * T
    x2 = x.reshape(rows, F)

    info = plsc.get_sparse_core_info()
    NC, NS = info.num_cores, info.num_subcores
    NW = NC * NS
    rpw = rows // NW
    chunk = 128
    n_chunks = rpw // chunk

    mesh = plsc.VectorSubcoreMesh(core_axis_name="c", subcore_axis_name="s")

    @functools.partial(
        pl.kernel,
        mesh=mesh,
        out_type=jax.ShapeDtypeStruct((rows, F), x.dtype),
        scratch_types=[
            pltpu.VMEM((chunk, F), jnp.float32),
            pltpu.VMEM((chunk, F), jnp.float32),
            pltpu.SemaphoreType.DMA,
            pltpu.SemaphoreType.DMA,
        ],
    )
    def sc_copy(x_hbm, out_hbm, buf0, buf1, sem0, sem1):
        wid = lax.axis_index("s") * NC + lax.axis_index("c")
        base = wid * rpw
        bufs = (buf0, buf1)
        sems = (sem0, sem1)

        def out_copy(i):
            off = base + i * chunk
            return pltpu.make_async_copy(
                bufs[i % 2], out_hbm.at[pl.ds(off, chunk), :], sems[i % 2]
            )

        for i in range(n_chunks):
            if i >= 2:
                out_copy(i - 2).wait()
            off = base + i * chunk
            pltpu.sync_copy(x_hbm.at[pl.ds(off, chunk), :], bufs[i % 2])
            out_copy(i).start()
        out_copy(n_chunks - 2).wait()
        out_copy(n_chunks - 1).wait()

    return sc_copy(x2).reshape(B, T, F)
